# speedup vs baseline: 1.1647x; 1.1647x over previous
import functools

import jax
import jax.numpy as jnp
from jax import lax
from jax.experimental import pallas as pl
from jax.experimental.pallas import tpu as pltpu
from jax.experimental.pallas import tpu_sc as plsc

N = 10000
K = 32
D = 128

_info = plsc.get_sparse_core_info()
_NC, _NS, _L = _info.num_cores, _info.num_subcores, _info.num_lanes
_NW = _NC * _NS

N_PAD = 10240
NODES_PER_W = N_PAD // _NW
CHUNK_NODES = 4
EDGES_PER_CHUNK = CHUNK_NODES * K
CHUNKS_PER_W = NODES_PER_W // CHUNK_NODES
_NVR = D // _L


def _sc_gather_sum(table, idx_flat):
    mesh = plsc.VectorSubcoreMesh(core_axis_name="c", subcore_axis_name="s")

    @functools.partial(
        pl.kernel,
        mesh=mesh,
        out_type=jax.ShapeDtypeStruct((N_PAD, D), jnp.float32),
        scratch_types=[
            pltpu.VMEM((EDGES_PER_CHUNK,), jnp.int32),
            pltpu.VMEM((EDGES_PER_CHUNK, D), jnp.float32),
            pltpu.VMEM((CHUNK_NODES, D), jnp.float32),
            pltpu.SemaphoreType.DMA,
        ],
    )
    def k(table_hbm, idx_hbm, out_hbm, idx_v, rows_v, acc_v, sem):
        wid = lax.axis_index("s") * _NC + lax.axis_index("c")
        ebase = wid * NODES_PER_W * K
        nbase = wid * NODES_PER_W

        def chunk_body(c, carry):
            pltpu.sync_copy(
                idx_hbm.at[pl.ds(ebase + c * EDGES_PER_CHUNK, EDGES_PER_CHUNK)],
                idx_v,
            )
            pltpu.async_copy(table_hbm.at[idx_v], rows_v, sem).wait()
            for i in range(CHUNK_NODES):
                def rbody(r, accs):
                    return tuple(
                        accs[v] + rows_v[i * K + r, pl.ds(v * _L, _L)]
                        for v in range(_NVR)
                    )
                accs = lax.fori_loop(
                    0, K, rbody,
                    tuple(jnp.zeros((_L,), jnp.float32) for _ in range(_NVR)),
                )
                for v in range(_NVR):
                    acc_v[i, pl.ds(v * _L, _L)] = accs[v]
            pltpu.sync_copy(
                acc_v, out_hbm.at[pl.ds(nbase + c * CHUNK_NODES, CHUNK_NODES)]
            )
            return carry

        lax.fori_loop(0, CHUNKS_PER_W, chunk_body, 0)

    return k(table, idx_flat)


_TC_BLK = 1000


def _tc_body(nf_ref, agg_ref, w_ref, b_ref, g_ref, be_ref, out_ref):
    t = lax.dot_general(
        agg_ref[...], w_ref[...],
        (((1,), (1,)), ((), ())),
        preferred_element_type=jnp.float32,
    )
    comb = nf_ref[...] + t + b_ref[...]
    mu = jnp.mean(comb, axis=-1, keepdims=True)
    dev = comb - mu
    var = jnp.mean(dev * dev, axis=-1, keepdims=True)
    normed = dev * lax.rsqrt(var + 1e-5) * g_ref[...] + be_ref[...]
    out_ref[...] = jnp.maximum(normed, 0.0)


def _tc_tail(node_features, agg, Ws, b, gamma, beta):
    grid = (N // _TC_BLK,)
    row_spec = pl.BlockSpec((_TC_BLK, D), lambda i: (i, 0))
    full_spec = pl.BlockSpec((D, D), lambda i: (0, 0))
    vec_spec = pl.BlockSpec((1, D), lambda i: (0, 0))
    return pl.pallas_call(
        _tc_body,
        grid=grid,
        in_specs=[row_spec, row_spec, full_spec, vec_spec, vec_spec, vec_spec],
        out_specs=row_spec,
        out_shape=jax.ShapeDtypeStruct((N, D), jnp.float32),
    )(node_features, agg, Ws,
      b.reshape(1, D), gamma.reshape(1, D), beta.reshape(1, D))


def kernel(node_features, neighbor_idx, W, b, gamma, beta):
    idx_flat = jnp.pad(neighbor_idx, ((0, N_PAD - N), (0, 0))).reshape(-1)
    aggsum = _sc_gather_sum(node_features, idx_flat)
    return _tc_tail(node_features, aggsum[:N], W / K, b, gamma, beta)


# speedup vs baseline: 1.3644x; 1.1715x over previous
import functools

import jax
import jax.numpy as jnp
from jax import lax
from jax.experimental import pallas as pl
from jax.experimental.pallas import tpu as pltpu
from jax.experimental.pallas import tpu_sc as plsc

N = 10000
K = 32
D = 128

_info = plsc.get_sparse_core_info()
_NC, _NS, _L = _info.num_cores, _info.num_subcores, _info.num_lanes
_NW = _NC * _NS

N_PAD = 10240
NODES_PER_W = N_PAD // _NW
CHUNK_NODES = 4
EDGES_PER_CHUNK = CHUNK_NODES * K
CHUNKS_PER_W = NODES_PER_W // CHUNK_NODES
_NVR = D // _L


_NBUF = 2


def _sc_gather_sum(table, idx_flat):
    mesh = plsc.VectorSubcoreMesh(core_axis_name="c", subcore_axis_name="s")

    @functools.partial(
        pl.kernel,
        mesh=mesh,
        out_type=jax.ShapeDtypeStruct((N_PAD, D), jnp.float32),
        scratch_types=[
            pltpu.VMEM((NODES_PER_W * K,), jnp.int32),
            pltpu.VMEM((_NBUF, EDGES_PER_CHUNK, D), jnp.float32),
            pltpu.VMEM((NODES_PER_W, D), jnp.float32),
            pltpu.SemaphoreType.DMA,
            pltpu.SemaphoreType.DMA,
        ],
    )
    def k(table_hbm, idx_hbm, out_hbm, idx_v, rows_v, out_stage, sem0, sem1):
        wid = lax.axis_index("s") * _NC + lax.axis_index("c")
        ebase = wid * NODES_PER_W * K
        nbase = wid * NODES_PER_W
        sems = (sem0, sem1)

        pltpu.sync_copy(idx_hbm.at[pl.ds(ebase, NODES_PER_W * K)], idx_v)

        def gather_start(c, b):
            pltpu.async_copy(
                table_hbm.at[idx_v.at[pl.ds(c * EDGES_PER_CHUNK, EDGES_PER_CHUNK)]],
                rows_v.at[b],
                sems[b],
            )

        def gather_wait(b):
            pltpu.make_async_copy(
                table_hbm.at[idx_v.at[pl.ds(0, EDGES_PER_CHUNK)]],
                rows_v.at[b],
                sems[b],
            ).wait()

        def reduce_chunk(c, b):
            for i in range(CHUNK_NODES):
                def rbody(r, accs):
                    base = i * K + 2 * r
                    return tuple(
                        accs[v]
                        + rows_v[b, base, pl.ds(v * _L, _L)]
                        + rows_v[b, base + 1, pl.ds(v * _L, _L)]
                        for v in range(_NVR)
                    )
                accs = lax.fori_loop(
                    0, K // 2, rbody,
                    tuple(jnp.zeros((_L,), jnp.float32) for _ in range(_NVR)),
                )
                for v in range(_NVR):
                    out_stage[c * CHUNK_NODES + i, pl.ds(v * _L, _L)] = accs[v]

        for b in range(_NBUF):
            gather_start(b, b)

        def group_body(g, carry):
            for b in range(_NBUF):
                c = g * _NBUF + b
                gather_wait(b)
                reduce_chunk(c, b)

                @pl.when(c + _NBUF < CHUNKS_PER_W)
                def _():
                    gather_start(c + _NBUF, b)
            return carry

        lax.fori_loop(0, CHUNKS_PER_W // _NBUF, group_body, 0)
        pltpu.sync_copy(out_stage, out_hbm.at[pl.ds(nbase, NODES_PER_W)])

    return k(table, idx_flat)


_TC_BLK = 1000


def _tc_body(nf_ref, agg_ref, w_ref, b_ref, g_ref, be_ref, out_ref):
    t = lax.dot_general(
        agg_ref[...], w_ref[...],
        (((1,), (1,)), ((), ())),
        preferred_element_type=jnp.float32,
    )
    comb = nf_ref[...] + t + b_ref[...]
    mu = jnp.mean(comb, axis=-1, keepdims=True)
    dev = comb - mu
    var = jnp.mean(dev * dev, axis=-1, keepdims=True)
    normed = dev * lax.rsqrt(var + 1e-5) * g_ref[...] + be_ref[...]
    out_ref[...] = jnp.maximum(normed, 0.0)


def _tc_tail(node_features, agg, Ws, b, gamma, beta):
    grid = (N // _TC_BLK,)
    row_spec = pl.BlockSpec((_TC_BLK, D), lambda i: (i, 0))
    full_spec = pl.BlockSpec((D, D), lambda i: (0, 0))
    vec_spec = pl.BlockSpec((1, D), lambda i: (0, 0))
    return pl.pallas_call(
        _tc_body,
        grid=grid,
        in_specs=[row_spec, row_spec, full_spec, vec_spec, vec_spec, vec_spec],
        out_specs=row_spec,
        out_shape=jax.ShapeDtypeStruct((N, D), jnp.float32),
    )(node_features, agg, Ws,
      b.reshape(1, D), gamma.reshape(1, D), beta.reshape(1, D))


def kernel(node_features, neighbor_idx, W, b, gamma, beta):
    idx_flat = jnp.pad(neighbor_idx, ((0, N_PAD - N), (0, 0))).reshape(-1)
    aggsum = _sc_gather_sum(node_features, idx_flat)
    return _tc_tail(node_features, aggsum[:N], W / K, b, gamma, beta)
